# baseline (speedup 1.0000x reference)
import functools

import jax
import jax.numpy as jnp
from jax import lax
from jax.experimental import pallas as pl
from jax.experimental.pallas import tpu as pltpu
from jax.experimental.pallas import tpu_sc as plsc

NSUB = 16
NCORE = 2
LANES = 16
CHUNK = 128
NBUF = 4


def _make_sc_scatter(n_acc, dh, nch):
    rows_per_sub = n_acc // NSUB
    mesh = plsc.VectorSubcoreMesh(core_axis_name="c", subcore_axis_name="s")

    @functools.partial(
        pl.kernel,
        mesh=mesh,
        compiler_params=pltpu.CompilerParams(use_tc_tiling_on_sc=False),
        out_type=jax.ShapeDtypeStruct((NCORE, n_acc, dh), jnp.float32),
        scratch_types=[
            pltpu.VMEM((nch, CHUNK), jnp.int32),
            pltpu.VMEM((nch, CHUNK), jnp.int32),
            *[pltpu.VMEM((CHUNK * LANES,), jnp.float32) for _ in range(NBUF)],
            *[pltpu.VMEM((CHUNK, dh), jnp.float32) for _ in range(NBUF)],
            pltpu.VMEM_SHARED((n_acc, dh), jnp.float32),
            *[pltpu.SemaphoreType.DMA for _ in range(2 * NBUF)],
        ],
    )
    def sc_scatter(f2_hbm, src_hbm, dst_hbm, w_hbm, out_hbm,
                   src_v, dst_v, w0, w1, w2, w3, b0, b1, b2, b3, acc,
                   g0, g1, g2, g3, s0, s1, s2, s3):
        c = lax.axis_index("c")
        s = lax.axis_index("s")
        bufs = (b0, b1, b2, b3)
        wbufs = (w0, w1, w2, w3)
        gsems = (g0, g1, g2, g3)
        ssems = (s0, s1, s2, s3)

        pltpu.sync_copy(src_hbm.at[c, s], src_v)
        pltpu.sync_copy(dst_hbm.at[s], dst_v)

        @plsc.parallel_loop(0, CHUNK, 1, unroll=8)
        def _zrow(r):
            for j in range(dh // LANES):
                bufs[0][r, pl.ds(LANES * j, LANES)] = jnp.zeros((LANES,), jnp.float32)

        for t in range(rows_per_sub // CHUNK):
            r0 = s * rows_per_sub + t * CHUNK
            pltpu.sync_copy(bufs[0], acc.at[pl.ds(r0, CHUNK)])
        plsc.subcore_barrier()

        def _start_gather(p, k):
            pltpu.async_copy(f2_hbm.at[src_v.at[p]], bufs[k], gsems[k])
            pltpu.async_copy(w_hbm.at[s * nch + p], wbufs[k], gsems[k])

        def _scale(buf, wbuf):
            @plsc.parallel_loop(0, CHUNK, 1, unroll=8)
            def _edge(e):
                wvec = wbuf[pl.ds(LANES * e, LANES)]
                for j in range(dh // LANES):
                    sl = pl.ds(LANES * j, LANES)
                    buf[e, sl] = buf[e, sl] * wvec

        def _drain_rows(sem, k):
            pltpu.make_async_copy(f2_hbm.at[pl.ds(0, CHUNK)], bufs[k], sem).wait()

        _start_gather(0, 0)
        _start_gather(1, 1)

        def _quad(i, carry):
            for k in range(NBUF):
                p = NBUF * i + k
                k2 = (k + 2) % NBUF
                _drain_rows(gsems[k], k)
                pltpu.make_async_copy(w_hbm.at[0], wbufs[k], gsems[k]).wait()
                _scale(bufs[k], wbufs[k])
                pltpu.async_copy(bufs[k], acc.at[dst_v.at[p]], ssems[k], add=True)

                @pl.when(p + 2 < nch)
                def _():
                    @pl.when(p >= 2)
                    def _():
                        _drain_rows(ssems[k2], k2)
                    _start_gather(p + 2, k2)
            return carry

        lax.fori_loop(0, nch // NBUF, _quad, 0)
        for k in range(NBUF):
            _drain_rows(ssems[k], k)

        plsc.subcore_barrier()
        for t in range(rows_per_sub // CHUNK):
            r0 = s * rows_per_sub + t * CHUNK
            pltpu.sync_copy(acc.at[pl.ds(r0, CHUNK)], bufs[0])
            pltpu.sync_copy(bufs[0], out_hbm.at[c, pl.ds(r0, CHUNK)])

    return sc_scatter


def _tc_linear(partial, W, b8, n_nodes):
    dh = partial.shape[2]
    d = 2 * dh
    blk = 1000 if n_nodes % 1000 == 0 else n_nodes

    def _body(p_ref, w_ref, b_ref, o_ref):
        y = lax.dot_general(p_ref[0], w_ref[:, 0:dh], (((1,), (1,)), ((), ())),
                            preferred_element_type=jnp.float32)
        y += lax.dot_general(p_ref[1], w_ref[:, dh:d], (((1,), (1,)), ((), ())),
                             preferred_element_type=jnp.float32)
        o_ref[...] = jnp.maximum(y + b_ref[0:1, :], 0.0)

    return pl.pallas_call(
        _body,
        grid=(n_nodes // blk,),
        in_specs=[
            pl.BlockSpec((2, blk, dh), lambda i: (0, i, 0)),
            pl.BlockSpec((d, d), lambda i: (0, 0)),
            pl.BlockSpec((8, d), lambda i: (0, 0)),
        ],
        out_specs=pl.BlockSpec((blk, d), lambda i: (i, 0)),
        out_shape=jax.ShapeDtypeStruct((n_nodes, d), jnp.float32),
    )(partial, W, b8)


def kernel(feature, edge_index, edge_weight, W, b):
    n_nodes, d = feature.shape
    dh = d // 2
    e = edge_index.shape[1]
    per_s = NSUB * CHUNK
    e_pad = ((e + per_s - 1) // per_s) * per_s
    nch = e_pad // per_s
    if nch % NBUF:
        nch += NBUF - nch % NBUF
        e_pad = nch * per_s

    src = edge_index[0].astype(jnp.int32)
    dst = edge_index[1].astype(jnp.int32)
    w = edge_weight.astype(jnp.float32)
    pad = e_pad - e
    src = jnp.concatenate([src, jnp.zeros((pad,), jnp.int32)])
    dst = jnp.concatenate([dst, jnp.zeros((pad,), jnp.int32)]).reshape(NSUB, nch, CHUNK)
    w = jnp.concatenate([w, jnp.zeros((pad,), jnp.float32)])
    f2 = jnp.concatenate([feature[:, :dh], feature[:, dh:]], axis=0)
    src2 = jnp.stack([src, src + n_nodes]).reshape(NCORE, NSUB, nch, CHUNK)
    w16 = jnp.broadcast_to(w[:, None], (e_pad, LANES)).reshape(NSUB * nch, CHUNK * LANES)

    n_acc = ((n_nodes + NSUB * CHUNK - 1) // (NSUB * CHUNK)) * (NSUB * CHUNK)
    partial = _make_sc_scatter(n_acc, dh, nch)(f2, src2, dst, w16)
    b8 = jnp.broadcast_to(b[None, :], (8, d))
    return _tc_linear(partial, W, b8, n_nodes)


# speedup vs baseline: 1.2502x; 1.2502x over previous
import functools

import jax
import jax.numpy as jnp
from jax import lax
from jax.experimental import pallas as pl
from jax.experimental.pallas import tpu as pltpu
from jax.experimental.pallas import tpu_sc as plsc

NSUB = 16
NCORE = 2
LANES = 16
CHUNK = 128
NBUF = 2


def _make_sc_scatter(n_acc, dh, nch):
    rows_per_sub = n_acc // NSUB
    mesh = plsc.VectorSubcoreMesh(core_axis_name="c", subcore_axis_name="s")

    @functools.partial(
        pl.kernel,
        mesh=mesh,
        compiler_params=pltpu.CompilerParams(use_tc_tiling_on_sc=False),
        out_type=jax.ShapeDtypeStruct((NCORE, n_acc, dh), jnp.float32),
        scratch_types=[
            pltpu.VMEM((nch, CHUNK), jnp.int32),
            pltpu.VMEM((nch, CHUNK), jnp.int32),
            *[pltpu.VMEM((CHUNK * LANES,), jnp.float32) for _ in range(NBUF)],
            *[pltpu.VMEM((CHUNK, dh), jnp.float32) for _ in range(NBUF)],
            pltpu.VMEM_SHARED((n_acc, dh), jnp.float32),
            *[pltpu.SemaphoreType.DMA for _ in range(NBUF)],
        ],
    )
    def sc_scatter(f2_hbm, src_hbm, dst_hbm, w_hbm, out_hbm,
                   src_v, dst_v, w0, w1, b0, b1, acc, g0, g1):
        c = lax.axis_index("c")
        s = lax.axis_index("s")
        bufs = (b0, b1)
        wbufs = (w0, w1)
        gsems = (g0, g1)

        pltpu.sync_copy(src_hbm.at[c, s], src_v)
        pltpu.sync_copy(dst_hbm.at[s], dst_v)

        @plsc.parallel_loop(0, CHUNK, 1, unroll=8)
        def _zrow(r):
            for j in range(dh // LANES):
                bufs[0][r, pl.ds(LANES * j, LANES)] = jnp.zeros((LANES,), jnp.float32)

        for t in range(rows_per_sub // CHUNK):
            r0 = s * rows_per_sub + t * CHUNK
            pltpu.sync_copy(bufs[0], acc.at[pl.ds(r0, CHUNK)])
        plsc.subcore_barrier()

        def _start_gather(p, k):
            pltpu.async_copy(f2_hbm.at[src_v.at[p]], bufs[k], gsems[k])
            pltpu.async_copy(w_hbm.at[s * nch + p], wbufs[k], gsems[k])

        def _scale(buf, wbuf):
            @plsc.parallel_loop(0, CHUNK, 1, unroll=8)
            def _edge(e):
                wvec = wbuf[pl.ds(LANES * e, LANES)]
                for j in range(dh // LANES):
                    sl = pl.ds(LANES * j, LANES)
                    buf[e, sl] = buf[e, sl] * wvec

        def _drain_rows(sem, k):
            pltpu.make_async_copy(f2_hbm.at[pl.ds(0, CHUNK)], bufs[k], sem).wait()

        _start_gather(0, 0)
        _start_gather(1, 1)

        def _pair(i, carry):
            for k in range(2):
                p = 2 * i + k
                _drain_rows(gsems[k], k)
                pltpu.make_async_copy(w_hbm.at[0], wbufs[k], gsems[k]).wait()
                _scale(bufs[k], wbufs[k])
                pltpu.sync_copy(bufs[k], acc.at[dst_v.at[p]], add=True)

                @pl.when(p + 2 < nch)
                def _():
                    _start_gather(p + 2, k)
            return carry

        lax.fori_loop(0, nch // 2, _pair, 0)

        plsc.subcore_barrier()
        for t in range(rows_per_sub // CHUNK):
            r0 = s * rows_per_sub + t * CHUNK
            pltpu.sync_copy(acc.at[pl.ds(r0, CHUNK)], bufs[0])
            pltpu.sync_copy(bufs[0], out_hbm.at[c, pl.ds(r0, CHUNK)])

    return sc_scatter


def _tc_linear(partial, W, b8, n_nodes):
    dh = partial.shape[2]
    d = 2 * dh
    blk = 1000 if n_nodes % 1000 == 0 else n_nodes

    def _body(p_ref, w_ref, b_ref, o_ref):
        y = lax.dot_general(p_ref[0], w_ref[:, 0:dh], (((1,), (1,)), ((), ())),
                            preferred_element_type=jnp.float32)
        y += lax.dot_general(p_ref[1], w_ref[:, dh:d], (((1,), (1,)), ((), ())),
                             preferred_element_type=jnp.float32)
        o_ref[...] = jnp.maximum(y + b_ref[0:1, :], 0.0)

    return pl.pallas_call(
        _body,
        grid=(n_nodes // blk,),
        in_specs=[
            pl.BlockSpec((2, blk, dh), lambda i: (0, i, 0)),
            pl.BlockSpec((d, d), lambda i: (0, 0)),
            pl.BlockSpec((8, d), lambda i: (0, 0)),
        ],
        out_specs=pl.BlockSpec((blk, d), lambda i: (i, 0)),
        out_shape=jax.ShapeDtypeStruct((n_nodes, d), jnp.float32),
    )(partial, W, b8)


def kernel(feature, edge_index, edge_weight, W, b):
    n_nodes, d = feature.shape
    dh = d // 2
    e = edge_index.shape[1]
    per_s = NSUB * CHUNK
    e_pad = ((e + per_s - 1) // per_s) * per_s
    nch = e_pad // per_s
    if nch % NBUF:
        nch += NBUF - nch % NBUF
        e_pad = nch * per_s

    src = edge_index[0].astype(jnp.int32)
    dst = edge_index[1].astype(jnp.int32)
    w = edge_weight.astype(jnp.float32)
    pad = e_pad - e
    src = jnp.concatenate([src, jnp.zeros((pad,), jnp.int32)])
    dst = jnp.concatenate([dst, jnp.zeros((pad,), jnp.int32)]).reshape(NSUB, nch, CHUNK)
    w = jnp.concatenate([w, jnp.zeros((pad,), jnp.float32)])
    f2 = jnp.concatenate([feature[:, :dh], feature[:, dh:]], axis=0)
    src2 = jnp.stack([src, src + n_nodes]).reshape(NCORE, NSUB, nch, CHUNK)
    w16 = jnp.broadcast_to(w[:, None], (e_pad, LANES)).reshape(NSUB * nch, CHUNK * LANES)

    n_acc = ((n_nodes + NSUB * CHUNK - 1) // (NSUB * CHUNK)) * (NSUB * CHUNK)
    partial = _make_sc_scatter(n_acc, dh, nch)(f2, src2, dst, w16)
    b8 = jnp.broadcast_to(b[None, :], (8, d))
    return _tc_linear(partial, W, b8, n_nodes)
